# Initial kernel scaffold; baseline (speedup 1.0000x reference)
import jax
import jax.numpy as jnp
from jax.experimental import pallas as pl


def kernel(x, pos, batch, params):
    raise NotImplementedError("write your pallas kernel here")


# speedup vs baseline: 3.7424x; 3.7424x over previous
import functools

import jax
import jax.numpy as jnp
from jax import lax
from jax.experimental import pallas as pl

K_NN = 30
N_POINTS = 8192
ROW_BLK = 256
N_BLKS = N_POINTS // ROW_BLK
BIG_MASK = 1e37


def _knn_uv_body(f_blk, f_full, brow, bcol, wu, wv, b1, idx_out, u_out, v_out):
    f = f_blk[...]
    fa = f_full[...]
    g = lax.dot_general(f.astype(jnp.bfloat16), fa.astype(jnp.bfloat16),
                        (((1,), (1,)), ((), ())),
                        preferred_element_type=jnp.float32)
    d2r = jnp.sum(f * f, axis=1, keepdims=True)
    ones = jnp.ones((1, f.shape[1]), jnp.float32)
    d2c = lax.dot_general(ones, fa * fa, (((1,), (1,)), ((), ())),
                          preferred_element_type=jnp.float32,
                          precision=lax.Precision.HIGHEST)
    dist = d2r + d2c - 2.0 * g
    dist = jnp.where(brow[...] != bcol[...], BIG_MASK, dist)
    col = lax.broadcasted_iota(jnp.int32, dist.shape, 1)
    big_i = jnp.int32(2**30)
    for kk in range(K_NN):
        m = jnp.min(dist, axis=1, keepdims=True)
        amin = jnp.min(jnp.where(dist == m, col, big_i), axis=1, keepdims=True)
        idx_out[:, kk:kk + 1] = amin
        dist = jnp.where(col == amin, jnp.inf, dist)
    u_out[...] = lax.dot_general(f, wu[...], (((1,), (0,)), ((), ())),
                                 preferred_element_type=jnp.float32) + b1[...]
    v_out[...] = lax.dot_general(f, wv[...], (((1,), (0,)), ((), ())),
                                 preferred_element_type=jnp.float32)


def _knn_uv(f, brow, bcol, wu, wv, b1):
    n, d = f.shape
    grid = (N_BLKS,)
    return pl.pallas_call(
        _knn_uv_body,
        grid=grid,
        in_specs=[
            pl.BlockSpec((ROW_BLK, d), lambda i: (i, 0)),
            pl.BlockSpec((n, d), lambda i: (0, 0)),
            pl.BlockSpec((ROW_BLK, 1), lambda i: (i, 0)),
            pl.BlockSpec((1, n), lambda i: (0, 0)),
            pl.BlockSpec((d, 64), lambda i: (0, 0)),
            pl.BlockSpec((d, 64), lambda i: (0, 0)),
            pl.BlockSpec((1, 64), lambda i: (0, 0)),
        ],
        out_specs=[
            pl.BlockSpec((ROW_BLK, K_NN), lambda i: (i, 0)),
            pl.BlockSpec((ROW_BLK, 64), lambda i: (i, 0)),
            pl.BlockSpec((ROW_BLK, 64), lambda i: (i, 0)),
        ],
        out_shape=[
            jax.ShapeDtypeStruct((n, K_NN), jnp.int32),
            jax.ShapeDtypeStruct((n, 64), jnp.float32),
            jax.ShapeDtypeStruct((n, 64), jnp.float32),
        ],
    )(f, f, brow, bcol, wu, wv, b1)


def _edge_conv(p, f, brow, bcol):
    d = f.shape[1]
    w1 = p["lin1"]["W"]
    wa, wb = w1[:d], w1[d:]
    wu = wa - wb
    b1 = p["lin1"]["b"].reshape(1, 64)
    idx, u, v = _knn_uv(f, brow, bcol, wu, wb, b1)
    xj = jnp.take(f, idx, axis=0)
    xi = jnp.broadcast_to(f[:, None, :], xj.shape)
    e = jnp.concatenate([xi, xj - xi], axis=-1)
    N, K, D = e.shape
    h = e.reshape(N * K, D) @ w1 + p["lin1"]["b"]
    m = jnp.mean(h, axis=0)
    var = jnp.var(h, axis=0)
    h = (h - m) / jnp.sqrt(var + 1e-5) * p["gamma"] + p["beta"]
    h = jax.nn.relu(h)
    h = h @ p["lin2"]["W"] + p["lin2"]["b"]
    return jnp.max(h.reshape(N, K, -1), axis=1)


def kernel(x, pos, batch, params):
    b32 = batch.astype(jnp.int32)
    brow = b32.reshape(N_POINTS, 1)
    bcol = b32.reshape(1, N_POINTS)
    x0 = jnp.concatenate([pos, x], axis=-1)
    x1 = _edge_conv(params["conv1"], x0, brow, bcol)
    x2 = _edge_conv(params["conv2"], x1, brow, bcol)
    x3 = _edge_conv(params["conv3"], x2, brow, bcol)
    h = jnp.concatenate([x1, x2, x3], axis=1)
    for p in params["head"][:-1]:
        h = jax.nn.relu(h @ p["W"] + p["b"])
    p = params["head"][-1]
    out = h @ p["W"] + p["b"]
    return (jax.nn.log_softmax(out, axis=1), x3)


# speedup vs baseline: 4.6044x; 1.2303x over previous
import functools

import jax
import jax.numpy as jnp
from jax import lax
from jax.experimental import pallas as pl
from jax.experimental.pallas import tpu as pltpu
from jax.experimental.pallas import tpu_sc as plsc

K_NN = 30
N_POINTS = 8192
ROW_BLK = 256
N_BLKS = N_POINTS // ROW_BLK
N_EDGES = N_POINTS * K_NN
BIG_MASK = 1e37


def _knn_body(f_blk, f_full, brow, bcol, idx_out):
    f = f_blk[...]
    fa = f_full[...]
    g = lax.dot_general(f.astype(jnp.bfloat16), fa.astype(jnp.bfloat16),
                        (((1,), (1,)), ((), ())),
                        preferred_element_type=jnp.float32)
    d2r = jnp.sum(f * f, axis=1, keepdims=True)
    ones = jnp.ones((1, f.shape[1]), jnp.float32)
    d2c = lax.dot_general(ones, fa * fa, (((1,), (1,)), ((), ())),
                          preferred_element_type=jnp.float32,
                          precision=lax.Precision.HIGHEST)
    dist = d2r + d2c - 2.0 * g
    dist = jnp.where(brow[...] != bcol[...], BIG_MASK, dist)
    col = lax.broadcasted_iota(jnp.int32, dist.shape, 1)
    big_i = jnp.int32(2**30)
    for kk in range(K_NN):
        m = jnp.min(dist, axis=1, keepdims=True)
        amin = jnp.min(jnp.where(dist == m, col, big_i), axis=1, keepdims=True)
        idx_out[:, kk:kk + 1] = amin
        dist = jnp.where(col == amin, jnp.inf, dist)


def _knn(f, brow, bcol):
    n, d = f.shape
    return pl.pallas_call(
        _knn_body,
        grid=(N_BLKS,),
        in_specs=[
            pl.BlockSpec((ROW_BLK, d), lambda i: (i, 0)),
            pl.BlockSpec((n, d), lambda i: (0, 0)),
            pl.BlockSpec((ROW_BLK, 1), lambda i: (i, 0)),
            pl.BlockSpec((1, n), lambda i: (0, 0)),
        ],
        out_specs=pl.BlockSpec((ROW_BLK, K_NN), lambda i: (i, 0)),
        out_shape=jax.ShapeDtypeStruct((n, K_NN), jnp.int32),
    )(f, f, brow, bcol)


GATHER_WIN = 128


def _sc_gather(f_pad, idx_flat):
    n, dp = f_pad.shape
    e = idx_flat.shape[1]
    mesh = plsc.VectorSubcoreMesh(core_axis_name="c", subcore_axis_name="s")

    @functools.partial(
        pl.kernel,
        out_type=jax.ShapeDtypeStruct((e, dp), jnp.float32),
        mesh=mesh,
        compiler_params=pltpu.CompilerParams(use_tc_tiling_on_sc=False),
    )
    def gather_kernel(x_hbm, i_hbm, o_hbm):
        def body(i_vmem, o_vmem):
            pltpu.sync_copy(x_hbm.at[i_vmem.at[0]], o_vmem)

        pltpu.emit_pipeline(
            body,
            grid=(e // GATHER_WIN,),
            in_specs=[pl.BlockSpec((1, GATHER_WIN), lambda i: (0, i))],
            out_specs=[pl.BlockSpec((GATHER_WIN, dp), lambda i: (i, 0))],
            core_axis_name=("c", "s"),
            dimension_semantics=(pltpu.PARALLEL,),
        )(i_hbm, o_hbm)

    return gather_kernel(f_pad, idx_flat)


def _conv_body(xj, f_blk, w1, b1, gamma, beta, w2, b2, out, acc):
    p = pl.program_id(0)
    b = pl.program_id(1)
    d = f_blk.shape[1]
    xi = f_blk[...][:, None, :]
    xjv = xj[...][:, :, :d]
    xib = jnp.broadcast_to(xi, xjv.shape)
    dn3 = (((2,), (0,)), ((), ()))
    s_a = lax.dot_general(xib.astype(jnp.bfloat16),
                          w1[...][:d].astype(jnp.bfloat16), dn3,
                          preferred_element_type=jnp.float32)
    s_b = lax.dot_general((xjv - xib).astype(jnp.bfloat16),
                          w1[...][d:].astype(jnp.bfloat16), dn3,
                          preferred_element_type=jnp.float32)
    h1 = (s_a + s_b) + b1[...]

    @pl.when((p == 0) & (b == 0))
    def _():
        acc[...] = jnp.zeros_like(acc)

    @pl.when(p == 0)
    def _():
        s1 = jnp.sum(jnp.sum(h1, axis=1), axis=0, keepdims=True)
        s2 = jnp.sum(jnp.sum(h1 * h1, axis=1), axis=0, keepdims=True)
        acc[0:1, :64] += s1
        acc[1:2, :64] += s2
        out[...] = jnp.zeros_like(out)

    @pl.when(p == 1)
    def _():
        cnt = jnp.float32(N_EDGES)
        m = acc[0:1, :64] / cnt
        var = acc[1:2, :64] / cnt - m * m
        hn = (h1 - m) / jnp.sqrt(var + 1e-5) * gamma[...] + beta[...]
        r = jax.nn.relu(hn)
        h2 = lax.dot_general(r.astype(jnp.bfloat16),
                             w2[...].astype(jnp.bfloat16),
                             (((2,), (0,)), ((), ())),
                             preferred_element_type=jnp.float32)
        out[...] = jnp.max(h2, axis=1) + b2[...]


def _conv_apply(p, xj3, f):
    d = f.shape[1]
    dp = xj3.shape[2]
    w1 = p["lin1"]["W"]
    b1 = p["lin1"]["b"].reshape(1, 64)
    gamma = p["gamma"].reshape(1, 64)
    beta = p["beta"].reshape(1, 64)
    w2 = p["lin2"]["W"]
    b2 = p["lin2"]["b"].reshape(1, 64)
    return pl.pallas_call(
        _conv_body,
        grid=(2, N_BLKS),
        in_specs=[
            pl.BlockSpec((ROW_BLK, K_NN, dp), lambda p_, b_: (b_, 0, 0)),
            pl.BlockSpec((ROW_BLK, d), lambda p_, b_: (b_, 0)),
            pl.BlockSpec((2 * d, 64), lambda p_, b_: (0, 0)),
            pl.BlockSpec((1, 64), lambda p_, b_: (0, 0)),
            pl.BlockSpec((1, 64), lambda p_, b_: (0, 0)),
            pl.BlockSpec((1, 64), lambda p_, b_: (0, 0)),
            pl.BlockSpec((64, 64), lambda p_, b_: (0, 0)),
            pl.BlockSpec((1, 64), lambda p_, b_: (0, 0)),
        ],
        out_specs=pl.BlockSpec((ROW_BLK, 64), lambda p_, b_: (b_, 0)),
        out_shape=jax.ShapeDtypeStruct((N_POINTS, 64), jnp.float32),
        scratch_shapes=[pltpu.VMEM((8, 128), jnp.float32)],
    )(xj3, f, w1, b1, gamma, beta, w2, b2)


def _edge_conv(p, f, f_pad, brow, bcol):
    dp = f_pad.shape[1]
    idx = _knn(f, brow, bcol)
    xj = _sc_gather(f_pad, idx.reshape(1, N_EDGES))
    xj3 = xj.reshape(N_POINTS, K_NN, dp)
    return _conv_apply(p, xj3, f)


def _edge_conv1_mlp(p, f, xj):
    d = f.shape[1]
    xi = jnp.broadcast_to(f[:, None, :], xj.shape)
    e = jnp.concatenate([xi, xj - xi], axis=-1)
    nk, dd = N_EDGES, 2 * d
    h = e.reshape(nk, dd) @ p["lin1"]["W"] + p["lin1"]["b"]
    m = jnp.mean(h, axis=0)
    var = jnp.var(h, axis=0)
    h = (h - m) / jnp.sqrt(var + 1e-5) * p["gamma"] + p["beta"]
    h = jax.nn.relu(h)
    h = h @ p["lin2"]["W"] + p["lin2"]["b"]
    return jnp.max(h.reshape(N_POINTS, K_NN, -1), axis=1)


HEAD_BLK = 512


def _head_body(h, w0, b0, w1, b1, w2, b2, w3, b3, out):
    def lin(a, w, bb):
        return lax.dot_general(a.astype(jnp.bfloat16),
                               w[...].astype(jnp.bfloat16),
                               (((1,), (0,)), ((), ())),
                               preferred_element_type=jnp.float32) + bb[...]

    a = jax.nn.relu(lin(h[...], w0, b0))
    a = jax.nn.relu(lin(a, w1, b1))
    a = jax.nn.relu(lin(a, w2, b2))
    a = lin(a, w3, b3)
    m = jnp.max(a, axis=1, keepdims=True)
    ex = jnp.exp(a - m)
    lse = jnp.log(jnp.sum(ex, axis=1, keepdims=True))
    out[...] = a - m - lse


def _head(params, h):
    ws = [p["W"] for p in params]
    bs = [p["b"].reshape(1, -1) for p in params]
    specs = []
    args = []
    for w, bb in zip(ws, bs):
        specs.append(pl.BlockSpec(w.shape, lambda p_: (0, 0)))
        specs.append(pl.BlockSpec(bb.shape, lambda p_: (0, 0)))
        args.extend([w, bb])
    return pl.pallas_call(
        _head_body,
        grid=(N_POINTS // HEAD_BLK,),
        in_specs=[pl.BlockSpec((HEAD_BLK, h.shape[1]), lambda p_: (p_, 0))] + specs,
        out_specs=pl.BlockSpec((HEAD_BLK, 50), lambda p_: (p_, 0)),
        out_shape=jax.ShapeDtypeStruct((N_POINTS, 50), jnp.float32),
    )(h, *args)


def kernel(x, pos, batch, params):
    b32 = batch.astype(jnp.int32)
    brow = b32.reshape(N_POINTS, 1)
    bcol = b32.reshape(1, N_POINTS)
    x0 = jnp.concatenate([pos, x], axis=-1)
    x0_pad = jnp.pad(x0, ((0, 0), (0, 10)))
    idx1 = _knn(x0, brow, bcol)
    xj1 = _sc_gather(x0_pad, idx1.reshape(1, N_EDGES))
    xj1 = xj1.reshape(N_POINTS, K_NN, 16)[:, :, :6]
    x1 = _edge_conv1_mlp(params["conv1"], x0, xj1)
    x2 = _edge_conv(params["conv2"], x1, x1, brow, bcol)
    x3 = _edge_conv(params["conv3"], x2, x2, brow, bcol)
    h = jnp.concatenate([x1, x2, x3], axis=1)
    return (_head(params["head"], h), x3)
